# Initial kernel scaffold; baseline (speedup 1.0000x reference)
import jax
import jax.numpy as jnp
from jax.experimental import pallas as pl


def kernel(input_ids, weight):
    raise NotImplementedError("write your pallas kernel here")


# speedup vs baseline: 1.8304x; 1.8304x over previous
import functools

import jax
import jax.numpy as jnp
from jax import lax
from jax.experimental import pallas as pl
from jax.experimental.pallas import tpu as pltpu
from jax.experimental.pallas import tpu_sc as plsc

NUM_EMBEDDINGS = 1000000
EMBEDDING_DIM = 64
BATCH = 16384 * 50

_INFO = plsc.get_sparse_core_info()
_NC = _INFO.num_cores
_NS = _INFO.num_subcores
_NW = _NC * _NS
_B_PER_W = BATCH // _NW
_CHUNK = 512
_STEPS = _B_PER_W // _CHUNK


def _make_gather():
    mesh = plsc.VectorSubcoreMesh(core_axis_name="c", subcore_axis_name="s")

    @functools.partial(
        pl.kernel,
        mesh=mesh,
        out_type=jax.ShapeDtypeStruct((BATCH, EMBEDDING_DIM), jnp.float32),
        scratch_types=[
            pltpu.VMEM((_B_PER_W,), jnp.int32),
            pltpu.VMEM((_CHUNK, EMBEDDING_DIM), jnp.float32),
            pltpu.SemaphoreType.DMA,
        ],
        compiler_params=pltpu.CompilerParams(use_tc_tiling_on_sc=False),
    )
    def gather_kernel(idx_hbm, table_hbm, out_hbm, idx_v, rows_v, sem):
        wid = lax.axis_index("s") * _NC + lax.axis_index("c")
        base = wid * _B_PER_W
        pltpu.sync_copy(idx_hbm.at[pl.ds(base, _B_PER_W)], idx_v)

        def step(i, carry):
            off = i * _CHUNK
            pltpu.async_copy(
                table_hbm.at[idx_v.at[pl.ds(off, _CHUNK)]], rows_v, sem
            ).wait()
            pltpu.sync_copy(rows_v, out_hbm.at[pl.ds(base + off, _CHUNK)])
            return carry

        lax.fori_loop(0, _STEPS, step, 0)

    return gather_kernel


_gather = _make_gather()


@jax.jit
def kernel(input_ids, weight):
    flat_ids = input_ids.reshape(BATCH)
    out = _gather(flat_ids, weight)
    return out.reshape(input_ids.shape[0], input_ids.shape[1], EMBEDDING_DIM)


# speedup vs baseline: 1.8665x; 1.0197x over previous
import functools

import jax
import jax.numpy as jnp
from jax import lax
from jax.experimental import pallas as pl
from jax.experimental.pallas import tpu as pltpu
from jax.experimental.pallas import tpu_sc as plsc

NUM_EMBEDDINGS = 1000000
EMBEDDING_DIM = 64
BATCH = 16384 * 50

_INFO = plsc.get_sparse_core_info()
_NC = _INFO.num_cores
_NS = _INFO.num_subcores
_NW = _NC * _NS
_B_PER_W = BATCH // _NW
_CHUNK = 400
_NBUF = 4
_STEPS = _B_PER_W // _CHUNK
_GROUPS = _STEPS // _NBUF


def _make_gather():
    mesh = plsc.VectorSubcoreMesh(core_axis_name="c", subcore_axis_name="s")

    @functools.partial(
        pl.kernel,
        mesh=mesh,
        out_type=jax.ShapeDtypeStruct((BATCH, EMBEDDING_DIM), jnp.float32),
        scratch_types=[
            pltpu.VMEM((_B_PER_W,), jnp.int32),
            [pltpu.VMEM((_CHUNK, EMBEDDING_DIM), jnp.float32)] * _NBUF,
            [pltpu.SemaphoreType.DMA] * _NBUF,
            [pltpu.SemaphoreType.DMA] * _NBUF,
        ],
        compiler_params=pltpu.CompilerParams(use_tc_tiling_on_sc=False),
    )
    def gather_kernel(idx_hbm, table_hbm, out_hbm, idx_v, rows, gsems, osems):
        wid = lax.axis_index("s") * _NC + lax.axis_index("c")
        base = wid * _B_PER_W
        pltpu.sync_copy(idx_hbm.at[pl.ds(base, _B_PER_W)], idx_v)

        def g_start(b, off):
            pltpu.async_copy(
                table_hbm.at[idx_v.at[pl.ds(off, _CHUNK)]], rows[b], gsems[b]
            )

        def g_wait(b, off):
            pltpu.make_async_copy(
                table_hbm.at[idx_v.at[pl.ds(off, _CHUNK)]], rows[b], gsems[b]
            ).wait()

        def o_start(b, off):
            pltpu.async_copy(
                rows[b], out_hbm.at[pl.ds(base + off, _CHUNK)], osems[b]
            )

        def o_wait(b, off):
            pltpu.make_async_copy(
                rows[b], out_hbm.at[pl.ds(base + off, _CHUNK)], osems[b]
            ).wait()

        for b in range(_NBUF):
            g_start(b, b * _CHUNK)

        def group(j, carry):
            off0 = j * _NBUF * _CHUNK
            for b in range(_NBUF):
                off = off0 + b * _CHUNK
                g_wait(b, off)
                o_start(b, off)
            for b in range(_NBUF):
                off = off0 + b * _CHUNK
                o_wait(b, off)
                g_start(b, off + _NBUF * _CHUNK)
            return carry

        lax.fori_loop(0, _GROUPS - 1, group, 0)

        off0 = (_GROUPS - 1) * _NBUF * _CHUNK
        for b in range(_NBUF):
            off = off0 + b * _CHUNK
            g_wait(b, off)
            o_start(b, off)
        for b in range(_NBUF):
            off = off0 + b * _CHUNK
            o_wait(b, off)

    return gather_kernel


_gather = _make_gather()


@jax.jit
def kernel(input_ids, weight):
    flat_ids = input_ids.reshape(BATCH)
    out = _gather(flat_ids, weight)
    return out.reshape(input_ids.shape[0], input_ids.shape[1], EMBEDDING_DIM)
